# baseline (speedup 1.0000x reference)
import functools

import jax
import jax.numpy as jnp
from jax import lax
from jax.experimental import pallas as pl
from jax.experimental.pallas import tpu as pltpu

_EPS = 1e-5


def _pick_block(n):
    for blk in (10000, 4000, 2048, 2000, 1024, 1000, 512, 496, 256):
        if n % blk == 0:
            return blk
    return n


def _mlp_kernel(x_hbm, W1_ref, b1_ref, g1_ref, be1_ref, W2_ref,
                g2_ref, be2_ref, out_ref, br_ref, xc_ref, xbuf_ref, sem,
                *, nb, blk, inv_n):
    d = W1_ref.shape[0]

    def copy_in(slot, i):
        return pltpu.make_async_copy(
            x_hbm.at[pl.ds(i * blk, blk), :], xbuf_ref.at[slot],
            sem.at[slot])

    copy_in(0, 0).start()
    copy_in(1, 1).start()

    def p0(i, carry):
        s1, q1 = carry
        slot = lax.rem(i, 3)

        @pl.when(i + 2 < nb)
        def _prefetch():
            copy_in(lax.rem(i + 2, 3), i + 2).start()

        copy_in(slot, i).wait()
        xb = xbuf_ref[slot]
        s1 = s1 + jnp.sum(xb, axis=0, keepdims=True)
        q1 = q1 + jnp.sum(xb * xb, axis=0, keepdims=True)
        xc_ref[pl.ds(i * blk, blk), :] = xb.astype(jnp.bfloat16)
        return s1, q1

    zrow = jnp.zeros((1, d), jnp.float32)
    s1, q1 = lax.fori_loop(0, nb, p0, (zrow, zrow))

    def bn_affine(s, q, g_ref, be_ref):
        mean = s * inv_n
        var = q * inv_n - mean * mean
        a = g_ref[...] * lax.rsqrt(var + _EPS)
        c = be_ref[...] - mean * a
        return a, c

    a1, c1 = bn_affine(s1, q1, g1_ref, be1_ref)
    W1s = (W1_ref[...] * a1).astype(jnp.bfloat16)
    bias1 = lax.dot_general(c1, W1_ref[...], (((1,), (1,)), ((), ())),
                            preferred_element_type=jnp.float32) + b1_ref[...]

    ones_bf = jnp.ones((1, blk), jnp.bfloat16)
    eye = jnp.eye(d, dtype=jnp.float32)

    def p1(i, carry):
        s2, q2 = carry
        xcb = xc_ref[pl.ds(i * blk, blk), :]
        z = lax.dot_general(xcb, W1s, (((1,), (1,)), ((), ())),
                            preferred_element_type=jnp.float32)
        h_bf = jnp.maximum(z + bias1, 0.0).astype(jnp.bfloat16)
        s2 = s2 + lax.dot_general(ones_bf, h_bf, (((1,), (0,)), ((), ())),
                                  preferred_element_type=jnp.float32)
        gram = lax.dot_general(h_bf, h_bf, (((0,), (0,)), ((), ())),
                               preferred_element_type=jnp.float32)
        q2 = q2 + jnp.sum(gram * eye, axis=0, keepdims=True)
        xc_ref[pl.ds(i * blk, blk), :] = h_bf
        return s2, q2

    s2, q2 = lax.fori_loop(0, nb, p1, (zrow, zrow))

    a2, c2 = bn_affine(s2, q2, g2_ref, be2_ref)
    W2s = (W2_ref[...] * a2).astype(jnp.bfloat16)
    br_ref[...] = lax.dot_general(c2, W2_ref[...], (((1,), (1,)), ((), ())),
                                  preferred_element_type=jnp.float32)

    def p2(i, carry):
        hcb = xc_ref[pl.ds(i * blk, blk), :]
        out_t = lax.dot_general(W2s, hcb, (((1,), (1,)), ((), ())),
                                preferred_element_type=jnp.float32)
        out_ref[i, :, :] = out_t
        return carry

    lax.fori_loop(0, nb, p2, 0)


def kernel(causal, gamma1, beta1, W1, b1, gamma2, beta2, W2, b2):
    n, d = causal.shape
    d_out = W2.shape[0]
    blk = _pick_block(n)
    nb = n // blk

    row = lambda v: v.reshape(1, -1)
    vmem = pl.BlockSpec(memory_space=pltpu.MemorySpace.VMEM)

    fn = pl.pallas_call(
        functools.partial(_mlp_kernel, nb=nb, blk=blk, inv_n=1.0 / n),
        in_specs=[
            pl.BlockSpec(memory_space=pl.MemorySpace.ANY),
            vmem, vmem, vmem, vmem,
            vmem, vmem, vmem,
        ],
        out_specs=(vmem, vmem),
        out_shape=(jax.ShapeDtypeStruct((nb, d_out, blk), jnp.float32),
                   jax.ShapeDtypeStruct((1, d_out), jnp.float32)),
        scratch_shapes=[
            pltpu.VMEM((n, d), jnp.bfloat16),
            pltpu.VMEM((3, blk, d), jnp.float32),
            pltpu.SemaphoreType.DMA((3,)),
        ],
    )
    out3, brow = fn(causal, W1, row(b1), row(gamma1), row(beta1),
                    W2, row(gamma2), row(beta2))
    bias = brow + b2.reshape(1, -1)
    return out3.transpose(0, 2, 1).reshape(n, d_out) + bias


# speedup vs baseline: 1.4228x; 1.4228x over previous
import functools

import jax
import jax.numpy as jnp
from jax import lax
from jax.experimental import pallas as pl
from jax.experimental.pallas import tpu as pltpu

_EPS = 1e-5


def _pick_block(n):
    for blk in (10000, 4000, 2048, 2000, 1024, 1000, 512, 496, 256):
        if n % blk == 0:
            return blk
    return n


def _mlp_kernel(x_hbm, W1_ref, b1_ref, g1_ref, be1_ref, W2_ref,
                g2_ref, be2_ref, out_ref, br_ref, xc_ref, xbuf_ref, sem,
                *, nb, blk, inv_n):
    d = W1_ref.shape[0]

    def copy_in(slot, i):
        return pltpu.make_async_copy(
            x_hbm.at[pl.ds(i * blk, blk), :], xbuf_ref.at[slot],
            sem.at[slot])

    copy_in(0, 0).start()
    copy_in(1, 1).start()

    def p0(i, carry):
        s1, q1 = carry
        slot = lax.rem(i, 3)

        @pl.when(i + 2 < nb)
        def _prefetch():
            copy_in(lax.rem(i + 2, 3), i + 2).start()

        copy_in(slot, i).wait()
        xb = xbuf_ref[slot]
        s1 = s1 + jnp.sum(xb, axis=0, keepdims=True)
        q1 = q1 + jnp.sum(xb * xb, axis=0, keepdims=True)
        xc_ref[pl.ds(i * blk, blk), :] = xb.astype(jnp.bfloat16)
        return s1, q1

    zrow = jnp.zeros((1, d), jnp.float32)
    s1, q1 = lax.fori_loop(0, nb, p0, (zrow, zrow))

    def bn_affine(s, q, g_ref, be_ref):
        mean = s * inv_n
        var = q * inv_n - mean * mean
        a = g_ref[...] * lax.rsqrt(var + _EPS)
        c = be_ref[...] - mean * a
        return a, c

    a1, c1 = bn_affine(s1, q1, g1_ref, be1_ref)
    W1s = (W1_ref[...] * a1).astype(jnp.bfloat16)
    bias1 = lax.dot_general(c1, W1_ref[...], (((1,), (1,)), ((), ())),
                            preferred_element_type=jnp.float32) + b1_ref[...]

    def p1(i, carry):
        s2, q2 = carry
        xcb = xc_ref[pl.ds(i * blk, blk), :]
        z = lax.dot_general(xcb, W1s, (((1,), (1,)), ((), ())),
                            preferred_element_type=jnp.float32)
        h = jnp.maximum(z + bias1, 0.0)
        s2 = s2 + jnp.sum(h, axis=0, keepdims=True)
        q2 = q2 + jnp.sum(h * h, axis=0, keepdims=True)
        xc_ref[pl.ds(i * blk, blk), :] = h.astype(jnp.bfloat16)
        return s2, q2

    s2, q2 = lax.fori_loop(0, nb, p1, (zrow, zrow))

    a2, c2 = bn_affine(s2, q2, g2_ref, be2_ref)
    W2s = (W2_ref[...] * a2).astype(jnp.bfloat16)
    br_ref[...] = lax.dot_general(c2, W2_ref[...], (((1,), (1,)), ((), ())),
                                  preferred_element_type=jnp.float32)

    out_ref[...] = lax.dot_general(W2s, xc_ref[...],
                                   (((1,), (1,)), ((), ())),
                                   preferred_element_type=jnp.float32)


def kernel(causal, gamma1, beta1, W1, b1, gamma2, beta2, W2, b2):
    n, d = causal.shape
    d_out = W2.shape[0]
    blk = _pick_block(n)
    nb = n // blk

    row = lambda v: v.reshape(1, -1)
    vmem = pl.BlockSpec(memory_space=pltpu.MemorySpace.VMEM)

    fn = pl.pallas_call(
        functools.partial(_mlp_kernel, nb=nb, blk=blk, inv_n=1.0 / n),
        in_specs=[
            pl.BlockSpec(memory_space=pl.MemorySpace.ANY),
            vmem, vmem, vmem, vmem,
            vmem, vmem, vmem,
        ],
        out_specs=(vmem, vmem),
        out_shape=(jax.ShapeDtypeStruct((d_out, n), jnp.float32),
                   jax.ShapeDtypeStruct((1, d_out), jnp.float32)),
        scratch_shapes=[
            pltpu.VMEM((n, d), jnp.bfloat16),
            pltpu.VMEM((3, blk, d), jnp.float32),
            pltpu.SemaphoreType.DMA((3,)),
        ],
    )
    out_t, brow = fn(causal, W1, row(b1), row(gamma1), row(beta1),
                     W2, row(gamma2), row(beta2))
    bias = brow + b2.reshape(1, -1)
    return out_t.T + bias


# speedup vs baseline: 1.6912x; 1.1886x over previous
import functools

import jax
import jax.numpy as jnp
from jax import lax
from jax.experimental import pallas as pl
from jax.experimental.pallas import tpu as pltpu

_EPS = 1e-5


def _pick_block(n):
    for blk in (10000, 4000, 2048, 2000, 1024, 1000, 512, 496, 256):
        if n % blk == 0:
            return blk
    return n


def _mlp_kernel(x_hbm, W1_ref, b1_ref, g1_ref, be1_ref, W2_ref,
                g2_ref, be2_ref, out_ref, br_ref, xc_ref, xbuf_ref, sem,
                *, nb, blk, inv_n):
    d = W1_ref.shape[0]

    def copy_in(slot, i):
        return pltpu.make_async_copy(
            x_hbm.at[pl.ds(i * blk, blk), :], xbuf_ref.at[slot],
            sem.at[slot])

    copy_in(0, 0).start()
    copy_in(1, 1).start()

    zrow = jnp.zeros((1, d), jnp.float32)
    s1, q1 = zrow, zrow
    for i in range(nb):
        if i + 2 < nb:
            copy_in((i + 2) % 3, i + 2).start()
        copy_in(i % 3, i).wait()
        xb = xbuf_ref[i % 3]
        s1 = s1 + jnp.sum(xb, axis=0, keepdims=True)
        q1 = q1 + jnp.sum(xb * xb, axis=0, keepdims=True)
        xc_ref[i * blk:(i + 1) * blk, :] = xb.astype(jnp.bfloat16)

    def bn_affine(s, q, g_ref, be_ref):
        mean = s * inv_n
        var = q * inv_n - mean * mean
        a = g_ref[...] * lax.rsqrt(var + _EPS)
        c = be_ref[...] - mean * a
        return a, c

    a1, c1 = bn_affine(s1, q1, g1_ref, be1_ref)
    W1s = (W1_ref[...] * a1).astype(jnp.bfloat16)
    bias1 = lax.dot_general(c1, W1_ref[...], (((1,), (1,)), ((), ())),
                            preferred_element_type=jnp.float32) + b1_ref[...]

    s2, q2 = zrow, zrow
    for i in range(nb):
        xcb = xc_ref[i * blk:(i + 1) * blk, :]
        z = lax.dot_general(xcb, W1s, (((1,), (1,)), ((), ())),
                            preferred_element_type=jnp.float32)
        h = jnp.maximum(z + bias1, 0.0)
        s2 = s2 + jnp.sum(h, axis=0, keepdims=True)
        q2 = q2 + jnp.sum(h * h, axis=0, keepdims=True)
        xc_ref[i * blk:(i + 1) * blk, :] = h.astype(jnp.bfloat16)

    a2, c2 = bn_affine(s2, q2, g2_ref, be2_ref)
    W2s = (W2_ref[...] * a2).astype(jnp.bfloat16)
    br_ref[...] = lax.dot_general(c2, W2_ref[...], (((1,), (1,)), ((), ())),
                                  preferred_element_type=jnp.float32)

    out_ref[...] = lax.dot_general(W2s, xc_ref[...],
                                   (((1,), (1,)), ((), ())),
                                   preferred_element_type=jnp.float32)


def kernel(causal, gamma1, beta1, W1, b1, gamma2, beta2, W2, b2):
    n, d = causal.shape
    d_out = W2.shape[0]
    blk = _pick_block(n)
    nb = n // blk

    row = lambda v: v.reshape(1, -1)
    vmem = pl.BlockSpec(memory_space=pltpu.MemorySpace.VMEM)

    fn = pl.pallas_call(
        functools.partial(_mlp_kernel, nb=nb, blk=blk, inv_n=1.0 / n),
        in_specs=[
            pl.BlockSpec(memory_space=pl.MemorySpace.ANY),
            vmem, vmem, vmem, vmem,
            vmem, vmem, vmem,
        ],
        out_specs=(vmem, vmem),
        out_shape=(jax.ShapeDtypeStruct((d_out, n), jnp.float32),
                   jax.ShapeDtypeStruct((1, d_out), jnp.float32)),
        scratch_shapes=[
            pltpu.VMEM((n, d), jnp.bfloat16),
            pltpu.VMEM((3, blk, d), jnp.float32),
            pltpu.SemaphoreType.DMA((3,)),
        ],
    )
    out_t, brow = fn(causal, W1, row(b1), row(gamma1), row(beta1),
                     W2, row(gamma2), row(beta2))
    bias = brow + b2.reshape(1, -1)
    return out_t.T + bias


# speedup vs baseline: 1.6924x; 1.0007x over previous
import functools

import jax
import jax.numpy as jnp
from jax import lax
from jax.experimental import pallas as pl
from jax.experimental.pallas import tpu as pltpu

_EPS = 1e-5


def _pick_block(n):
    for blk in (10000, 4000, 2048, 2000, 1024, 1000, 512, 496, 256):
        if n % blk == 0:
            return blk
    return n


def _mlp_kernel(x_hbm, W1_ref, b1_ref, g1_ref, be1_ref, W2_ref,
                g2_ref, be2_ref, out_ref, br_ref, xc_ref, xbuf_ref, sem,
                *, nb, blk, inv_n):
    d = W1_ref.shape[0]

    def copy_in(slot, i):
        return pltpu.make_async_copy(
            x_hbm.at[pl.ds(i * blk, blk), :], xbuf_ref.at[slot],
            sem.at[slot])

    copy_in(0, 0).start()
    if nb > 1:
        copy_in(1, 1).start()

    zrow = jnp.zeros((1, d), jnp.float32)
    s1, q1 = zrow, zrow
    for i in range(nb):
        if i + 2 < nb:
            copy_in((i + 2) % 3, i + 2).start()
        copy_in(i % 3, i).wait()
        xb = xbuf_ref[i % 3]
        s1 = s1 + jnp.sum(xb, axis=0, keepdims=True)
        q1 = q1 + jnp.sum(xb * xb, axis=0, keepdims=True)
        xc_ref[i * blk:(i + 1) * blk, :] = xb.astype(jnp.bfloat16)

    def bn_affine(s, q, g_ref, be_ref):
        mean = s * inv_n
        var = q * inv_n - mean * mean
        a = g_ref[...] * lax.rsqrt(var + _EPS)
        c = be_ref[...] - mean * a
        return a, c

    a1, c1 = bn_affine(s1, q1, g1_ref, be1_ref)
    W1s = (W1_ref[...] * a1).astype(jnp.bfloat16)
    bias1 = lax.dot_general(c1, W1_ref[...], (((1,), (1,)), ((), ())),
                            preferred_element_type=jnp.float32) + b1_ref[...]

    s2, q2 = zrow, zrow
    for i in range(nb):
        xcb = xc_ref[i * blk:(i + 1) * blk, :]
        z = lax.dot_general(xcb, W1s, (((1,), (1,)), ((), ())),
                            preferred_element_type=jnp.float32)
        h = jnp.maximum(z + bias1, 0.0)
        s2 = s2 + jnp.sum(h, axis=0, keepdims=True)
        q2 = q2 + jnp.sum(h * h, axis=0, keepdims=True)
        xc_ref[i * blk:(i + 1) * blk, :] = h.astype(jnp.bfloat16)

    a2, c2 = bn_affine(s2, q2, g2_ref, be2_ref)
    W2s = (W2_ref[...] * a2).astype(jnp.bfloat16)
    br_ref[...] = lax.dot_general(c2, W2_ref[...], (((1,), (1,)), ((), ())),
                                  preferred_element_type=jnp.float32)

    out_ref[...] = lax.dot_general(W2s, xc_ref[...],
                                   (((1,), (1,)), ((), ())),
                                   preferred_element_type=jnp.float32)


def kernel(causal, gamma1, beta1, W1, b1, gamma2, beta2, W2, b2):
    n, d = causal.shape
    d_out = W2.shape[0]
    blk = _pick_block(n)
    nb = n // blk

    row = lambda v: v.reshape(1, -1)
    vmem = pl.BlockSpec(memory_space=pltpu.MemorySpace.VMEM)

    fn = pl.pallas_call(
        functools.partial(_mlp_kernel, nb=nb, blk=blk, inv_n=1.0 / n),
        in_specs=[
            pl.BlockSpec(memory_space=pl.MemorySpace.ANY),
            vmem, vmem, vmem, vmem,
            vmem, vmem, vmem,
        ],
        out_specs=(vmem, vmem),
        out_shape=(jax.ShapeDtypeStruct((d_out, n), jnp.float32),
                   jax.ShapeDtypeStruct((1, d_out), jnp.float32)),
        scratch_shapes=[
            pltpu.VMEM((n, d), jnp.bfloat16),
            pltpu.VMEM((3, blk, d), jnp.float32),
            pltpu.SemaphoreType.DMA((3,)),
        ],
    )
    out_t, brow = fn(causal, W1, row(b1), row(gamma1), row(beta1),
                     W2, row(gamma2), row(beta2))
    bias = brow + b2.reshape(1, -1)
    return out_t.T + bias
